# baseline (speedup 1.0000x reference)
import functools

import jax
import jax.numpy as jnp
from jax.experimental import pallas as pl
from jax.experimental.pallas import tpu as pltpu


_GAMMA = 0.1


def _kwta_kernel(x_ref, o_ref, key_ref, *, k):
    x = x_ref[...]
    bits = jax.lax.bitcast_convert_type(x, jnp.int32)
    key_ref[...] = bits ^ (
        jax.lax.shift_right_arithmetic(bits, 31) & jnp.int32(0x7FFFFFFF)
    )

    rows = x.shape[0]
    n = x.shape[1]
    n_seg = 32
    seg = n // n_seg

    def body(b, t):
        bit = jnp.left_shift(jnp.int32(1), jnp.int32(31) - b)
        cand = t + bit
        parts = [
            jnp.sum(
                (key_ref[:, g * seg:(g + 1) * seg] >= cand).astype(jnp.int32),
                axis=1,
                keepdims=True,
            )
            for g in range(n_seg)
        ]
        cnt = functools.reduce(jnp.add, parts)
        return jnp.where(cnt >= k, cand, t)

    t0 = jnp.full((rows, 1), jnp.iinfo(jnp.int32).min, jnp.int32)
    t = jax.lax.fori_loop(0, 32, body, t0)

    o_ref[...] = jnp.where(key_ref[...] >= t, jnp.float32(0.0), x_ref[...])


@jax.jit
def kernel(x):
    B, C, H, W = x.shape
    n = H * W
    k = int(_GAMMA * n)
    rows_total = B * C
    block_rows = 8
    assert rows_total % block_rows == 0
    x2 = x.reshape(rows_total, n)
    out = pl.pallas_call(
        functools.partial(_kwta_kernel, k=k),
        grid=(rows_total // block_rows,),
        in_specs=[pl.BlockSpec((block_rows, n), lambda i: (i, 0))],
        out_specs=pl.BlockSpec((block_rows, n), lambda i: (i, 0)),
        out_shape=jax.ShapeDtypeStruct((rows_total, n), x.dtype),
        scratch_shapes=[pltpu.VMEM((block_rows, n), jnp.int32)],
    )(x2)
    return out.reshape(B, C, H, W)


# speedup vs baseline: 1.0459x; 1.0459x over previous
import functools

import jax
import jax.numpy as jnp
from jax.experimental import pallas as pl
from jax.experimental.pallas import tpu as pltpu


_GAMMA = 0.1


def _kwta_kernel(x_ref, o_ref, key_ref, *, k):
    x = x_ref[...]
    bits = jax.lax.bitcast_convert_type(x, jnp.int32)
    key_ref[...] = bits ^ (
        jax.lax.shift_right_arithmetic(bits, 31) & jnp.int32(0x7FFFFFFF)
    )

    rows = x.shape[0]
    n = x.shape[1]
    n_seg = 16
    seg = n // n_seg

    def body(b, t):
        bit = jnp.left_shift(jnp.int32(1), jnp.int32(31) - b)
        cand = t + bit
        parts = [
            jnp.sum(
                (key_ref[:, g * seg:(g + 1) * seg] >= cand).astype(jnp.int32),
                axis=1,
                keepdims=True,
            )
            for g in range(n_seg)
        ]
        cnt = functools.reduce(jnp.add, parts)
        return jnp.where(cnt >= k, cand, t)

    t0 = jnp.full((rows, 1), jnp.iinfo(jnp.int32).min, jnp.int32)
    t = jax.lax.fori_loop(0, 32, body, t0)

    o_ref[...] = jnp.where(key_ref[...] >= t, jnp.float32(0.0), x_ref[...])


@jax.jit
def kernel(x):
    B, C, H, W = x.shape
    n = H * W
    k = int(_GAMMA * n)
    rows_total = B * C
    block_rows = 16
    assert rows_total % block_rows == 0
    x2 = x.reshape(rows_total, n)
    out = pl.pallas_call(
        functools.partial(_kwta_kernel, k=k),
        grid=(rows_total // block_rows,),
        in_specs=[pl.BlockSpec((block_rows, n), lambda i: (i, 0))],
        out_specs=pl.BlockSpec((block_rows, n), lambda i: (i, 0)),
        out_shape=jax.ShapeDtypeStruct((rows_total, n), x.dtype),
        scratch_shapes=[pltpu.VMEM((block_rows, n), jnp.int32)],
    )(x2)
    return out.reshape(B, C, H, W)
